# Initial kernel scaffold; baseline (speedup 1.0000x reference)
import jax
import jax.numpy as jnp
from jax.experimental import pallas as pl


def kernel(x, edge_index, place_idx, src_idx, dst_idx, W1, b1, W2, b2, Wd1, bd1, Wd2, bd2):
    raise NotImplementedError("write your pallas kernel here")


# speedup vs baseline: 3.0135x; 3.0135x over previous
import functools

import jax
import jax.numpy as jnp
from jax import lax
from jax.experimental import pallas as pl
from jax.experimental.pallas import tpu as pltpu
from jax.experimental.pallas import tpu_sc as plsc

N = 10000
N_PAD = 10240
E = 320000
F = 128
P = 8192

NC = 2
NS = 16
NW = NC * NS

CH = 128
NCH = 80
PER_W = NCH * CH
E_PAD = NW * PER_W

GTOT = 3 * P
G_PER_W = GTOT // NW
GC = 128
NG = G_PER_W // GC
NG_PAD = 8

ROWS_PER_TILE = N_PAD // NS

IDX_HALF = NCH // 2


@functools.lru_cache(maxsize=None)
def _mesh():
  return plsc.VectorSubcoreMesh(
      core_axis_name="c", subcore_axis_name="s", num_cores=NC, num_subcores=NS)

f32 = jnp.float32
i32 = jnp.int32


def _sc_edges_body(table, src_r, dst_r, zeros_hbm, agg_out,
                   srcw, dstw, rows0, rows1, acc, sem0, sem1):
  c = lax.axis_index("c")
  s = lax.axis_index("s")
  wid = c * NS + s
  rz = s * ROWS_PER_TILE

  pltpu.sync_copy(zeros_hbm, rows0)
  for k in range(ROWS_PER_TILE // CH):
    pltpu.sync_copy(rows0, acc.at[pl.ds(rz + k * CH, CH)])
  plsc.subcore_barrier()

  for h in range(2):
    pltpu.sync_copy(src_r.at[wid].at[pl.ds(h * IDX_HALF, IDX_HALF)], srcw)
    pltpu.sync_copy(dst_r.at[wid].at[pl.ds(h * IDX_HALF, IDX_HALF)], dstw)
    pltpu.async_copy(table.at[srcw.at[0]], rows0, sem0)

    def outer(i, carry):
      j0 = 2 * i
      pltpu.async_copy(table.at[srcw.at[j0 + 1]], rows1, sem1)
      pltpu.make_async_copy(table.at[srcw.at[j0]], rows0, sem0).wait()
      pltpu.sync_copy(rows0, acc.at[dstw.at[j0]], add=True)

      @pl.when(j0 + 2 < IDX_HALF)
      def _():
        pltpu.async_copy(table.at[srcw.at[j0 + 2]], rows0, sem0)

      pltpu.make_async_copy(table.at[srcw.at[j0 + 1]], rows1, sem1).wait()
      pltpu.sync_copy(rows1, acc.at[dstw.at[j0 + 1]], add=True)
      return carry

    lax.fori_loop(0, IDX_HALF // 2, outer, 0)

  plsc.subcore_barrier()
  pltpu.sync_copy(acc.at[pl.ds(rz, ROWS_PER_TILE)],
                  agg_out.at[c].at[pl.ds(rz, ROWS_PER_TILE)])


@functools.lru_cache(maxsize=None)
def _make_sc_edges():
  return pl.kernel(
      _sc_edges_body,
      out_type=jax.ShapeDtypeStruct((NC, N_PAD, F), f32),
      mesh=_mesh(),
      scratch_types=[
          pltpu.VMEM((IDX_HALF, CH), i32),
          pltpu.VMEM((IDX_HALF, CH), i32),
          pltpu.VMEM((CH, F), f32),
          pltpu.VMEM((CH, F), f32),
          pltpu.VMEM_SHARED((N_PAD, F), f32),
          pltpu.SemaphoreType.DMA,
          pltpu.SemaphoreType.DMA,
      ],
      name="sc_edge_agg",
  )


def _sc_edges(*args):
  return _make_sc_edges()(*args)


def _sc_gather_body(ftab, idx3_r, g_out, gidx, gbuf0, gbuf1, sem0, sem1):
  c = lax.axis_index("c")
  s = lax.axis_index("s")
  wid = c * NS + s
  base = wid * G_PER_W
  pltpu.sync_copy(idx3_r.at[wid], gidx)
  bufs = (gbuf0, gbuf1)
  sems = (sem0, sem1)
  pltpu.async_copy(ftab.at[gidx.at[0]], gbuf0, sem0)
  for k in range(NG):
    b = k % 2
    if k + 1 < NG:
      pltpu.async_copy(ftab.at[gidx.at[k + 1]], bufs[1 - b], sems[1 - b])
    pltpu.make_async_copy(ftab.at[gidx.at[k]], bufs[b], sems[b]).wait()
    pltpu.sync_copy(bufs[b], g_out.at[pl.ds(base + k * GC, GC)])


@functools.lru_cache(maxsize=None)
def _make_sc_gather():
  return pl.kernel(
      _sc_gather_body,
      out_type=jax.ShapeDtypeStruct((GTOT, F), f32),
      mesh=_mesh(),
      scratch_types=[
          pltpu.VMEM((NG_PAD, GC), i32),
          pltpu.VMEM((GC, F), f32),
          pltpu.VMEM((GC, F), f32),
          pltpu.SemaphoreType.DMA,
          pltpu.SemaphoreType.DMA,
      ],
      name="sc_gather3",
  )


def _sc_gather3(*args):
  return _make_sc_gather()(*args)


PRED_ROWS = 4 * P
PRED_PER_TILE = PRED_ROWS // NS
PNG = PRED_PER_TILE // CH


def _sc_preds_body(pred4, place_r, zeros_hbm, out, pidx, pbuf0, pbuf1, acc,
                   sem0, sem1):
  c = lax.axis_index("c")
  s = lax.axis_index("s")

  @pl.when(c == 0)
  def _():
    rz = s * ROWS_PER_TILE
    pltpu.sync_copy(zeros_hbm, pbuf0)
    for k in range(ROWS_PER_TILE // CH):
      pltpu.sync_copy(pbuf0, acc.at[pl.ds(rz + k * CH, CH)])
    pltpu.sync_copy(place_r.at[s], pidx)
    plsc.subcore_barrier()
    base = s * PRED_PER_TILE
    bufs = (pbuf0, pbuf1)
    sems = (sem0, sem1)
    pltpu.async_copy(pred4.at[pl.ds(base, CH)], pbuf0, sem0)
    for k in range(PNG):
      b = k % 2
      if k + 1 < PNG:
        pltpu.async_copy(pred4.at[pl.ds(base + (k + 1) * CH, CH)],
                         bufs[1 - b], sems[1 - b])
      pltpu.make_async_copy(pred4.at[pl.ds(base + k * CH, CH)],
                            bufs[b], sems[b]).wait()
      pltpu.sync_copy(bufs[b], acc.at[pidx.at[k]], add=True)
    plsc.subcore_barrier()
    pltpu.sync_copy(acc.at[pl.ds(rz, ROWS_PER_TILE)],
                    out.at[pl.ds(rz, ROWS_PER_TILE)])


@functools.lru_cache(maxsize=None)
def _make_sc_preds():
  return pl.kernel(
      _sc_preds_body,
      out_type=jax.ShapeDtypeStruct((N_PAD, F), f32),
      mesh=_mesh(),
      scratch_types=[
          pltpu.VMEM((PNG, CH), i32),
          pltpu.VMEM((CH, F), f32),
          pltpu.VMEM((CH, F), f32),
          pltpu.VMEM_SHARED((N_PAD, F), f32),
          pltpu.SemaphoreType.DMA,
          pltpu.SemaphoreType.DMA,
      ],
      name="sc_pred_scatter",
  )


def _sc_preds(*args):
  return _make_sc_preds()(*args)


NBLK = 1024
PBLK = 1024


def _mm_body(x_ref, w_ref, o_ref):
  o_ref[...] = jnp.dot(x_ref[...], w_ref[...], preferred_element_type=f32)


def _tc_matmul(x, w):
  return pl.pallas_call(
      _mm_body,
      grid=(N_PAD // NBLK,),
      in_specs=[
          pl.BlockSpec((NBLK, F), lambda i: (i, 0)),
          pl.BlockSpec((F, F), lambda i: (0, 0)),
      ],
      out_specs=pl.BlockSpec((NBLK, F), lambda i: (i, 0)),
      out_shape=jax.ShapeDtypeStruct((N_PAD, F), f32),
  )(x, w)


def _fuse_body_xw(p0_ref, p1_ref, b_ref, w_ref, f_ref, xw_ref):
  f = jnp.maximum(p0_ref[...] + p1_ref[...] + b_ref[...], 0.0)
  f_ref[...] = f
  xw_ref[...] = jnp.dot(f, w_ref[...], preferred_element_type=f32)


def _fuse_body(p0_ref, p1_ref, b_ref, f_ref):
  f_ref[...] = jnp.maximum(p0_ref[...] + p1_ref[...] + b_ref[...], 0.0)


def _tc_fuse(p0, p1, b, w=None):
  b2d = b.reshape(1, F)
  if w is None:
    return pl.pallas_call(
        _fuse_body,
        grid=(N_PAD // NBLK,),
        in_specs=[
            pl.BlockSpec((NBLK, F), lambda i: (i, 0)),
            pl.BlockSpec((NBLK, F), lambda i: (i, 0)),
            pl.BlockSpec((1, F), lambda i: (0, 0)),
        ],
        out_specs=pl.BlockSpec((NBLK, F), lambda i: (i, 0)),
        out_shape=jax.ShapeDtypeStruct((N_PAD, F), f32),
    )(p0, p1, b2d)
  return pl.pallas_call(
      _fuse_body_xw,
      grid=(N_PAD // NBLK,),
      in_specs=[
          pl.BlockSpec((NBLK, F), lambda i: (i, 0)),
          pl.BlockSpec((NBLK, F), lambda i: (i, 0)),
          pl.BlockSpec((1, F), lambda i: (0, 0)),
          pl.BlockSpec((F, F), lambda i: (0, 0)),
      ],
      out_specs=[
          pl.BlockSpec((NBLK, F), lambda i: (i, 0)),
          pl.BlockSpec((NBLK, F), lambda i: (i, 0)),
      ],
      out_shape=[
          jax.ShapeDtypeStruct((N_PAD, F), f32),
          jax.ShapeDtypeStruct((N_PAD, F), f32),
      ],
  )(p0, p1, b2d, w)


def _dec_body(pf_ref, sf_ref, df_ref, wd1_ref, bd1_ref, wd2_ref, bd2_ref,
              o_ref):
  wd1 = wd1_ref[...]
  h = (jnp.dot(pf_ref[...], wd1[0:F], preferred_element_type=f32)
       + jnp.dot(sf_ref[...], wd1[F:2 * F], preferred_element_type=f32)
       + jnp.dot(df_ref[...], wd1[2 * F:3 * F], preferred_element_type=f32)
       + bd1_ref[...])
  h = jnp.maximum(h, 0.0)
  z = jnp.dot(h, wd2_ref[...], preferred_element_type=f32) + bd2_ref[...]
  ls = jnp.where(z >= 0.0, -jnp.log1p(jnp.exp(-z)), z - jnp.log1p(jnp.exp(z)))
  col = lax.broadcasted_iota(i32, (PBLK, F), 1)
  o_ref[...] = jnp.where(col == 0, ls, 0.0)


def _tc_decoder(pf, sf, df, wd1, bd1, wd2p, bd2p):
  return pl.pallas_call(
      _dec_body,
      grid=(P // PBLK,),
      in_specs=[
          pl.BlockSpec((PBLK, F), lambda i: (i, 0)),
          pl.BlockSpec((PBLK, F), lambda i: (i, 0)),
          pl.BlockSpec((PBLK, F), lambda i: (i, 0)),
          pl.BlockSpec((3 * F, F), lambda i: (0, 0)),
          pl.BlockSpec((1, F), lambda i: (0, 0)),
          pl.BlockSpec((F, F), lambda i: (0, 0)),
          pl.BlockSpec((1, F), lambda i: (0, 0)),
      ],
      out_specs=pl.BlockSpec((PBLK, F), lambda i: (i, 0)),
      out_shape=jax.ShapeDtypeStruct((P, F), f32),
  )(pf, sf, df, wd1, bd1, wd2p, bd2p)


def kernel(x, edge_index, place_idx, src_idx, dst_idx,
           W1, b1, W2, b2, Wd1, bd1, Wd2, bd2):
  src_pad = jnp.concatenate(
      [edge_index[0].astype(i32), jnp.zeros((E_PAD - E,), i32)])
  dst_pad = jnp.concatenate(
      [edge_index[1].astype(i32), jnp.full((E_PAD - E,), N_PAD - 1, i32)])
  src_r = src_pad.reshape(NW, NCH, CH)
  dst_r = dst_pad.reshape(NW, NCH, CH)
  zeros_nf = jnp.zeros((CH, F), f32)
  wd2p = jnp.pad(Wd2, ((0, 0), (0, F - Wd2.shape[1])))
  bd2p = jnp.pad(bd2.reshape(1, -1), ((0, 0), (0, F - bd2.shape[0])))
  bd1_2d = bd1.reshape(1, F)

  def idx3_for(v):
    flat = jnp.concatenate(
        [place_idx[v], src_idx[v], dst_idx[v]]).reshape(NW, NG, GC).astype(i32)
    return jnp.pad(flat, ((0, 0), (0, NG_PAD - NG), (0, 0)))

  x_pad = jnp.pad(x, ((0, N_PAD - N), (0, 0)))
  xw = _tc_matmul(x_pad, W1)
  agg = _sc_edges(xw, src_r, dst_r, zeros_nf)
  f, xw = _tc_fuse(agg[0], agg[1], b1, W2)

  preds = []
  for v in range(4):
    if v < 3:
      agg = _sc_edges(xw, src_r, dst_r, zeros_nf)
    g3 = _sc_gather3(f, idx3_for(v))
    pred = _tc_decoder(g3[0:P], g3[P:2 * P], g3[2 * P:3 * P],
                       Wd1, bd1_2d, wd2p, bd2p)
    preds.append(pred)
    if v == 2:
      f = _tc_fuse(agg[0], agg[1], b2)
    elif v < 2:
      f, xw = _tc_fuse(agg[0], agg[1], b2, W2)

  pred4 = jnp.concatenate(preds, axis=0)
  place_r = place_idx.reshape(NS, PNG, CH).astype(i32)
  pout = _sc_preds(pred4, place_r, zeros_nf)
  return pout[:N, :1]
